# Initial kernel scaffold; baseline (speedup 1.0000x reference)
import jax
import jax.numpy as jnp
from jax.experimental import pallas as pl


def kernel(x, tables):
    raise NotImplementedError("write your pallas kernel here")


# speedup vs baseline: 14.2540x; 14.2540x over previous
import functools

import jax
import jax.numpy as jnp
from jax import lax
from jax.experimental import pallas as pl
from jax.experimental.pallas import tpu as pltpu
from jax.experimental.pallas import tpu_sc as plsc

ATTRS_ = (25, 6, 18, 3, 9, 6, 4, 5, 5, 3, 3, 3, 3, 3, 10)
DIMS_ = (10, 3, 9, 3, 5, 3, 2, 3, 3, 2, 2, 2, 2, 2, 5)
B_ = 16384
OUT_W = 1 + sum(DIMS_)

_XCOL = []
_FBASE = []
_fb = 0
for _i in range(15):
    for _d in range(DIMS_[_i]):
        _XCOL.append(_i + 1)
        _FBASE.append(_fb)
        _fb += ATTRS_[_i]
FLAT_LEN = _fb
FLAT_PAD = ((FLAT_LEN + 7) // 8) * 8

NW = 32
ROWS_PER_W = B_ // NW
L = 16
NBLK = ROWS_PER_W // L


def _body(x_hbm, flat_hbm, out_hbm, x_v, flat_v, out_v):
    wid = lax.axis_index("s") * 2 + lax.axis_index("c")
    base = wid * ROWS_PER_W
    pltpu.sync_copy(x_hbm.at[pl.ds(base, ROWS_PER_W)], x_v)
    pltpu.sync_copy(flat_hbm, flat_v)

    lane = lax.broadcasted_iota(jnp.int32, (L,), 0)

    def block(b, _):
        row_ids = b * L + lane
        colv = [
            plsc.load_gather(x_v, [row_ids, jnp.full((L,), j, jnp.int32)])
            for j in range(16)
        ]
        plsc.store_scatter(out_v, [row_ids, jnp.zeros((L,), jnp.int32)], colv[0])
        idxv = [None] + [colv[j].astype(jnp.int32) for j in range(1, 16)]
        for c in range(OUT_W - 1):
            v = plsc.load_gather(flat_v, [idxv[_XCOL[c]] + _FBASE[c]])
            plsc.store_scatter(
                out_v, [row_ids, jnp.full((L,), c + 1, jnp.int32)], v
            )
        return _

    lax.fori_loop(0, NBLK, block, None)
    pltpu.sync_copy(out_v, out_hbm.at[pl.ds(base, ROWS_PER_W)])


@functools.partial(jax.jit, static_argnames=("interpret",))
def kernel(x, tables, interpret=False):
    flat = jnp.concatenate(
        [tables[i][:, d] for i in range(15) for d in range(DIMS_[i])]
    )
    flat = jnp.pad(flat, (0, FLAT_PAD - FLAT_LEN))
    run = pl.kernel(
        _body,
        out_type=jax.ShapeDtypeStruct((B_, OUT_W), jnp.float32),
        mesh=plsc.VectorSubcoreMesh(
            core_axis_name="c", subcore_axis_name="s",
            num_cores=2, num_subcores=16,
        ),
        scratch_types=[
            pltpu.VMEM((ROWS_PER_W, 16), jnp.float32),
            pltpu.VMEM((FLAT_PAD,), jnp.float32),
            pltpu.VMEM((ROWS_PER_W, OUT_W), jnp.float32),
        ],
        compiler_params=pltpu.CompilerParams(
            needs_layout_passes=False, use_tc_tiling_on_sc=False
        ),
        interpret=interpret,
    )
    return run(x, flat)


# speedup vs baseline: 15.8268x; 1.1103x over previous
import functools

import jax
import jax.numpy as jnp
from jax import lax
from jax.experimental import pallas as pl
from jax.experimental.pallas import tpu as pltpu
from jax.experimental.pallas import tpu_sc as plsc

ATTRS_ = (25, 6, 18, 3, 9, 6, 4, 5, 5, 3, 3, 3, 3, 3, 10)
DIMS_ = (10, 3, 9, 3, 5, 3, 2, 3, 3, 2, 2, 2, 2, 2, 5)
B_ = 16384
OUT_W = 1 + sum(DIMS_)

_XCOL = []
_FBASE = []
_fb = 0
for _i in range(15):
    for _d in range(DIMS_[_i]):
        _XCOL.append(_i + 1)
        _FBASE.append(_fb)
        _fb += ATTRS_[_i]
FLAT_LEN = _fb
FLAT_PAD = ((FLAT_LEN + 7) // 8) * 8

NW = 32
ROWS_PER_W = B_ // NW
L = 16
CHUNK = 256
NCHUNK = ROWS_PER_W // CHUNK
NBLK = CHUNK // L


def _body(x_hbm, flat_hbm, out_hbm, x_v, flat_v, out_v):
    wid = lax.axis_index("s") * 2 + lax.axis_index("c")
    pltpu.sync_copy(flat_hbm, flat_v)

    lane = lax.broadcasted_iota(jnp.int32, (L,), 0)

    def chunk(ch, _):
        base = wid * ROWS_PER_W + ch * CHUNK
        pltpu.sync_copy(x_hbm.at[pl.ds(base, CHUNK)], x_v)

        def block(b, _):
            row_ids = b * L + lane
            colv = [
                plsc.load_gather(x_v, [row_ids, jnp.full((L,), j, jnp.int32)])
                for j in range(16)
            ]
            plsc.store_scatter(
                out_v, [row_ids, jnp.zeros((L,), jnp.int32)], colv[0]
            )
            idxv = [None] + [colv[j].astype(jnp.int32) for j in range(1, 16)]
            for c in range(OUT_W - 1):
                v = plsc.load_gather(flat_v, [idxv[_XCOL[c]] + _FBASE[c]])
                plsc.store_scatter(
                    out_v, [row_ids, jnp.full((L,), c + 1, jnp.int32)], v
                )
            return _

        lax.fori_loop(0, NBLK, block, None)
        pltpu.sync_copy(out_v, out_hbm.at[pl.ds(base, CHUNK)])
        return _

    lax.fori_loop(0, NCHUNK, chunk, None)


@functools.partial(jax.jit, static_argnames=("interpret",))
def kernel(x, tables, interpret=False):
    flat = jnp.concatenate(
        [tables[i][:, d] for i in range(15) for d in range(DIMS_[i])]
    )
    flat = jnp.pad(flat, (0, FLAT_PAD - FLAT_LEN))
    run = pl.kernel(
        _body,
        out_type=jax.ShapeDtypeStruct((B_, OUT_W), jnp.float32),
        mesh=plsc.VectorSubcoreMesh(
            core_axis_name="c", subcore_axis_name="s",
            num_cores=2, num_subcores=16,
        ),
        scratch_types=[
            pltpu.VMEM((CHUNK, 16), jnp.float32),
            pltpu.VMEM((FLAT_PAD,), jnp.float32),
            pltpu.VMEM((CHUNK, OUT_W), jnp.float32),
        ],
        compiler_params=pltpu.CompilerParams(
            needs_layout_passes=False, use_tc_tiling_on_sc=True
        ),
        interpret=interpret,
    )
    return run(x, flat)


# speedup vs baseline: 27.7473x; 1.7532x over previous
import functools

import jax
import jax.numpy as jnp
from jax import lax
from jax.experimental import pallas as pl
from jax.experimental.pallas import tpu as pltpu
from jax.experimental.pallas import tpu_sc as plsc

ATTRS_ = (25, 6, 18, 3, 9, 6, 4, 5, 5, 3, 3, 3, 3, 3, 10)
DIMS_ = (10, 3, 9, 3, 5, 3, 2, 3, 3, 2, 2, 2, 2, 2, 5)
B_ = 16384
OUT_W = 1 + sum(DIMS_)

_XCOL = []
_FBASE = []
_fb = 0
for _i in range(15):
    for _d in range(DIMS_[_i]):
        _XCOL.append(_i + 1)
        _FBASE.append(_fb)
        _fb += ATTRS_[_i]
FLAT_LEN = _fb
FLAT_PAD = ((FLAT_LEN + 7) // 8) * 8

NW = 32
ROWS_PER_W = B_ // NW
L = 16
NBLK = ROWS_PER_W // L


def _body(xt_hbm, flat_hbm, out_hbm, xt_v, flat_v, out_v):
    wid = lax.axis_index("s") * 2 + lax.axis_index("c")
    base = wid * ROWS_PER_W
    pltpu.sync_copy(flat_hbm, flat_v)
    pltpu.sync_copy(xt_hbm.at[:, pl.ds(base, ROWS_PER_W)], xt_v)

    def block(b, _):
        rr = b * L
        sl = pl.ds(rr, L)
        out_v[0, sl] = xt_v[0, sl]
        idxv = [None] + [xt_v[j, sl].astype(jnp.int32) for j in range(1, 16)]
        for c in range(OUT_W - 1):
            v = plsc.load_gather(flat_v, [idxv[_XCOL[c]] + _FBASE[c]])
            out_v[c + 1, sl] = v
        return _

    lax.fori_loop(0, NBLK, block, None)
    pltpu.sync_copy(out_v, out_hbm.at[:, pl.ds(base, ROWS_PER_W)])


@functools.partial(jax.jit, static_argnames=("interpret",))
def kernel(x, tables, interpret=False):
    flat = jnp.concatenate(
        [tables[i][:, d] for i in range(15) for d in range(DIMS_[i])]
    )
    flat = jnp.pad(flat, (0, FLAT_PAD - FLAT_LEN))
    run = pl.kernel(
        _body,
        out_type=jax.ShapeDtypeStruct((OUT_W, B_), jnp.float32),
        mesh=plsc.VectorSubcoreMesh(
            core_axis_name="c", subcore_axis_name="s",
            num_cores=2, num_subcores=16,
        ),
        scratch_types=[
            pltpu.VMEM((16, ROWS_PER_W), jnp.float32),
            pltpu.VMEM((FLAT_PAD,), jnp.float32),
            pltpu.VMEM((OUT_W, ROWS_PER_W), jnp.float32),
        ],
        compiler_params=pltpu.CompilerParams(
            needs_layout_passes=False, use_tc_tiling_on_sc=True
        ),
        interpret=interpret,
    )
    return run(x.T, flat).T
